# Initial kernel scaffold; baseline (speedup 1.0000x reference)
import jax
import jax.numpy as jnp
from jax.experimental import pallas as pl


def kernel(h, u, W, b):
    raise NotImplementedError("write your pallas kernel here")


# speedup vs baseline: 2.4358x; 2.4358x over previous
import functools

import jax
import jax.numpy as jnp
from jax.experimental import pallas as pl

_EMB = 1024
_E = 16
_BLK = 512


def _gate_kernel(h_ref, u_ref, W_ref, b_ref, o_ref):
    h = h_ref[...]
    u = u_ref[...]
    Wh = W_ref[:_EMB, :]
    Wu = W_ref[_EMB:, :]
    g = (
        jax.lax.dot(h, Wh, preferred_element_type=jnp.float32)
        + jax.lax.dot(u, Wu, preferred_element_type=jnp.float32)
        + b_ref[...]
    )
    m = jnp.max(g, axis=-1, keepdims=True)
    e = jnp.exp(g - m)
    w = e / jnp.sum(e, axis=-1, keepdims=True)

    iota = jax.lax.broadcasted_iota(jnp.int32, w.shape, 1)
    m1 = jnp.max(w, axis=-1, keepdims=True)
    i1 = jnp.min(jnp.where(w == m1, iota, _E), axis=-1, keepdims=True)
    w2 = jnp.where(iota == i1, -jnp.inf, w)
    m2 = jnp.max(w2, axis=-1, keepdims=True)
    i2 = jnp.min(jnp.where(w2 == m2, iota, _E), axis=-1, keepdims=True)
    keep = (iota == i1) | (iota == i2)
    denom = m1 + m2 + 1e-9
    o_ref[...] = jnp.where(keep, w / denom, 0.0)


@jax.jit
def kernel(h, u, W, b):
    n = h.shape[0]
    grid = (n // _BLK,)
    return pl.pallas_call(
        _gate_kernel,
        grid=grid,
        in_specs=[
            pl.BlockSpec((_BLK, _EMB), lambda i: (i, 0)),
            pl.BlockSpec((_BLK, u.shape[1]), lambda i: (i, 0)),
            pl.BlockSpec(W.shape, lambda i: (0, 0)),
            pl.BlockSpec(b.shape, lambda i: (0,)),
        ],
        out_specs=pl.BlockSpec((_BLK, _E), lambda i: (i, 0)),
        out_shape=jax.ShapeDtypeStruct((n, _E), jnp.float32),
    )(h, u, W, b)


# speedup vs baseline: 2.6232x; 1.0769x over previous
import functools

import jax
import jax.numpy as jnp
from jax.experimental import pallas as pl

_EMB = 1024
_E = 16
_BLK = 512


def _gate_kernel(h_ref, u_ref, W_ref, b_ref, o_ref):
    h = h_ref[...]
    u = u_ref[...]
    Wh = W_ref[:_EMB, :]
    Wu = W_ref[_EMB:, :]
    g = (
        jax.lax.dot(h, Wh, preferred_element_type=jnp.float32)
        + jax.lax.dot(u, Wu, preferred_element_type=jnp.float32)
        + b_ref[...]
    )
    m = jnp.max(g, axis=-1, keepdims=True)
    iota = jax.lax.broadcasted_iota(jnp.int32, g.shape, 1)
    i1 = jnp.min(jnp.where(g == m, iota, _E), axis=-1, keepdims=True)
    e = jnp.exp(g - m)
    e_rest = jnp.where(iota == i1, -1.0, e)
    e2 = jnp.max(e_rest, axis=-1, keepdims=True)
    S = jnp.sum(e, axis=-1, keepdims=True)
    r = 1.0 / (1.0 + e2 + 1e-9 * S)
    keep = (iota == i1) | (e_rest >= e2)
    o_ref[...] = jnp.where(keep, e * r, 0.0)


@jax.jit
def kernel(h, u, W, b):
    n = h.shape[0]
    grid = (n // _BLK,)
    return pl.pallas_call(
        _gate_kernel,
        grid=grid,
        in_specs=[
            pl.BlockSpec((_BLK, _EMB), lambda i: (i, 0)),
            pl.BlockSpec((_BLK, u.shape[1]), lambda i: (i, 0)),
            pl.BlockSpec(W.shape, lambda i: (0, 0)),
            pl.BlockSpec(b.shape, lambda i: (0,)),
        ],
        out_specs=pl.BlockSpec((_BLK, _E), lambda i: (i, 0)),
        out_shape=jax.ShapeDtypeStruct((n, _E), jnp.float32),
    )(h, u, W, b)


# speedup vs baseline: 3.0889x; 1.1775x over previous
import functools

import jax
import jax.numpy as jnp
from jax.experimental import pallas as pl

_EMB = 1024
_E = 16
_BLK = 1024


def _gate_kernel(h_ref, u_ref, W_ref, b_ref, o_ref):
    h = h_ref[...]
    u = u_ref[...]
    Wh = W_ref[:_EMB, :]
    Wu = W_ref[_EMB:, :]
    g = (
        jax.lax.dot(h, Wh, preferred_element_type=jnp.float32)
        + jax.lax.dot(u, Wu, preferred_element_type=jnp.float32)
        + b_ref[...]
    )
    m = jnp.max(g, axis=-1, keepdims=True)
    iota = jax.lax.broadcasted_iota(jnp.int32, g.shape, 1)
    i1 = jnp.min(jnp.where(g == m, iota, _E), axis=-1, keepdims=True)
    e = jnp.exp(g - m)
    e_rest = jnp.where(iota == i1, -1.0, e)
    e2 = jnp.max(e_rest, axis=-1, keepdims=True)
    S = jnp.sum(e, axis=-1, keepdims=True)
    r = 1.0 / (1.0 + e2 + 1e-9 * S)
    keep = (iota == i1) | (e_rest >= e2)
    o_ref[...] = jnp.where(keep, e * r, 0.0)


@jax.jit
def kernel(h, u, W, b):
    n = h.shape[0]
    grid = (n // _BLK,)
    return pl.pallas_call(
        _gate_kernel,
        grid=grid,
        in_specs=[
            pl.BlockSpec((_BLK, _EMB), lambda i: (i, 0)),
            pl.BlockSpec((_BLK, u.shape[1]), lambda i: (i, 0)),
            pl.BlockSpec(W.shape, lambda i: (0, 0)),
            pl.BlockSpec(b.shape, lambda i: (0,)),
        ],
        out_specs=pl.BlockSpec((_BLK, _E), lambda i: (i, 0)),
        out_shape=jax.ShapeDtypeStruct((n, _E), jnp.float32),
    )(h, u, W, b)


# speedup vs baseline: 3.2718x; 1.0592x over previous
import functools

import jax
import jax.numpy as jnp
from jax.experimental import pallas as pl

_EMB = 1024
_E = 16
_BLK = 2048


def _gate_kernel(h_ref, u_ref, W_ref, b_ref, o_ref):
    h = h_ref[...]
    u = u_ref[...]
    Wh = W_ref[:_EMB, :]
    Wu = W_ref[_EMB:, :]
    g = (
        jax.lax.dot(h, Wh, preferred_element_type=jnp.float32)
        + jax.lax.dot(u, Wu, preferred_element_type=jnp.float32)
        + b_ref[...]
    )
    m = jnp.max(g, axis=-1, keepdims=True)
    iota = jax.lax.broadcasted_iota(jnp.int32, g.shape, 1)
    i1 = jnp.min(jnp.where(g == m, iota, _E), axis=-1, keepdims=True)
    e = jnp.exp(g - m)
    e_rest = jnp.where(iota == i1, -1.0, e)
    e2 = jnp.max(e_rest, axis=-1, keepdims=True)
    S = jnp.sum(e, axis=-1, keepdims=True)
    r = 1.0 / (1.0 + e2 + 1e-9 * S)
    keep = (iota == i1) | (e_rest >= e2)
    o_ref[...] = jnp.where(keep, e * r, 0.0)


@jax.jit
def kernel(h, u, W, b):
    n = h.shape[0]
    grid = (n // _BLK,)
    return pl.pallas_call(
        _gate_kernel,
        grid=grid,
        in_specs=[
            pl.BlockSpec((_BLK, _EMB), lambda i: (i, 0)),
            pl.BlockSpec((_BLK, u.shape[1]), lambda i: (i, 0)),
            pl.BlockSpec(W.shape, lambda i: (0, 0)),
            pl.BlockSpec(b.shape, lambda i: (0,)),
        ],
        out_specs=pl.BlockSpec((_BLK, _E), lambda i: (i, 0)),
        out_shape=jax.ShapeDtypeStruct((n, _E), jnp.float32),
    )(h, u, W, b)
